# baseline (speedup 1.0000x reference)
import functools

import jax
import jax.numpy as jnp
from jax import lax
from jax.experimental import pallas as pl
from jax.experimental.pallas import tpu as pltpu
from jax.experimental.pallas import tpu_sc as plsc

N_ENTRIES = 1000000
PARAM_DIM = 64
BATCH = 16384
PAD = 2 * PARAM_DIM

NC = 2
NS = 16
NW = NC * NS
B_PER_W = BATCH // NW
CHUNK = 128
NCH = B_PER_W // CHUNK
L = 16

_mesh = plsc.VectorSubcoreMesh(core_axis_name="c", subcore_axis_name="s")


@functools.partial(
    pl.kernel,
    out_type=jax.ShapeDtypeStruct((BATCH, PAD), jnp.float32),
    mesh=_mesh,
    scratch_types=[
        pltpu.VMEM((B_PER_W,), jnp.int32),
        pltpu.VMEM((B_PER_W,), jnp.int32),
        pltpu.VMEM((B_PER_W,), jnp.int32),
        pltpu.VMEM((B_PER_W + 1, PAD), jnp.float32),
        pltpu.SemaphoreType.DMA,
    ],
    compiler_params=pltpu.CompilerParams(needs_layout_passes=False),
)
def _lookup_kernel(x_hbm, t2_hbm, out_hbm, idx_v, row_v, odd_v, buf_v, sem):
    wid = lax.axis_index("s") * NC + lax.axis_index("c")
    base = wid * B_PER_W

    pltpu.sync_copy(x_hbm.at[pl.ds(base, B_PER_W)], idx_v)

    cnt = 0
    for g in range(B_PER_W // L):
        sl = pl.ds(g * L, L)
        v = idx_v[sl]
        row_v[sl] = jax.lax.shift_right_logical(v, 1)
        odd = jax.lax.ne(jnp.bitwise_and(v, jnp.int32(1)), jnp.int32(0))
        k = jax.lax.iota(jnp.int32, L) + g * L
        plsc.store_compressed(odd_v.at[pl.ds(cnt, L)], k, mask=odd)
        n16 = plsc.all_reduce_population_count(odd)
        cnt = cnt + n16[0]
    n_odd = cnt

    copies = []
    for c in range(NCH):
        copies.append(
            pltpu.async_copy(
                t2_hbm.at[row_v.at[pl.ds(c * CHUNK, CHUNK)]],
                buf_v.at[pl.ds(c * CHUNK, CHUNK)],
                sem,
            )
        )
    for c in copies:
        c.wait()

    sent = jnp.full((L,), B_PER_W, jnp.int32)
    n_grp = jax.lax.div(n_odd + (L - 1), L)
    plsc.store_compressed(
        odd_v.at[pl.ds(n_odd, L)], sent, mask=jnp.full((L,), True, jnp.bool_)
    )

    def fix_group(j, _):
        k16 = odd_v[pl.ds(j * L, L)]
        for cstep in range(PARAM_DIM):
            src_c = jnp.full((L,), PARAM_DIM + cstep, jnp.int32)
            vals = plsc.load_gather(buf_v, [k16, src_c])
            plsc.store_scatter(buf_v, [k16, src_c - PARAM_DIM], vals)
        return 0

    lax.fori_loop(0, n_grp, fix_group, 0)

    pltpu.sync_copy(
        buf_v.at[pl.ds(0, B_PER_W)], out_hbm.at[pl.ds(base, B_PER_W)]
    )


def kernel(x, table):
    t2 = table.reshape(N_ENTRIES // 2, PAD)
    y = _lookup_kernel(x, t2)
    return y[:, :PARAM_DIM]
